# baseline (speedup 1.0000x reference)
import functools

import jax
import jax.numpy as jnp
from jax import lax
from jax.experimental import pallas as pl
from jax.experimental.pallas import tpu as pltpu
from jax.experimental.pallas import tpu_sc as plsc

F = 26
OUT = 16
B = 16384
NW = 32
BPT = B // NW
C = 128
NCHUNK = BPT // C
CF = C * F
PER = 208
NREP = CF // PER


def _field_linear_sc(xf, weight, opat, bias):
    mesh = plsc.VectorSubcoreMesh(core_axis_name="c", subcore_axis_name="s")

    @functools.partial(
        pl.kernel,
        out_type=jax.ShapeDtypeStruct((B, OUT), jnp.float32),
        mesh=mesh,
        compiler_params=pltpu.CompilerParams(use_tc_tiling_on_sc=False),
        scratch_types=[
            pltpu.VMEM((PER,), jnp.int32),
            pltpu.VMEM((OUT,), jnp.float32),
            pltpu.VMEM((2, CF), jnp.int32),
            pltpu.VMEM((2, CF), jnp.int32),
            pltpu.VMEM((2, CF, OUT), jnp.float32),
            pltpu.VMEM((C, OUT), jnp.float32),
            pltpu.SemaphoreType.DMA,
            pltpu.SemaphoreType.DMA,
        ],
    )
    def k(xf_hbm, w_hbm, opat_hbm, bias_hbm, out_hbm,
          opat_v, bias_v, xv, idx_v, gbuf, outb, sem0, sem1):
        cid = lax.axis_index("c")
        sid = lax.axis_index("s")
        wid = sid * 2 + cid
        tbase = wid * BPT
        sems = (sem0, sem1)

        pltpu.sync_copy(opat_hbm, opat_v)
        pltpu.sync_copy(bias_hbm, bias_v)
        bias_vec = bias_v[:]

        def stage_in(ci, pb):
            base = tbase + ci * C
            pltpu.sync_copy(xf_hbm.at[pl.ds(base * F, CF)], xv.at[pb])

            def rep_body(r, carry):
                v0 = r * PER
                for t in range(PER // 16):
                    s = pl.ds(v0 + t * 16, 16)
                    idx_v[pb, s] = xv[pb, s] + opat_v[pl.ds(t * 16, 16)]
                return carry

            lax.fori_loop(0, NREP, rep_body, 0)
            return [
                pltpu.async_copy(w_hbm.at[idx_v.at[pb, pl.ds(g * C, C)]],
                                 gbuf.at[pb, pl.ds(g * C, C), :], sems[pb])
                for g in range(F)
            ]

        def stage_out(ci, pb, descs):
            for dsc in descs:
                dsc.wait()

            def row_body(j, carry):
                rbase = j * F
                acc = bias_vec
                for f in range(F):
                    acc = acc + gbuf[pb, rbase + f, :]
                outb[j, :] = acc
                return carry

            lax.fori_loop(0, C, row_body, 0)
            base = tbase + ci * C
            pltpu.sync_copy(outb, out_hbm.at[pl.ds(base, C), :])

        descs = stage_in(0, 0)
        for ci in range(NCHUNK):
            nxt = None
            if ci + 1 < NCHUNK:
                nxt = stage_in(ci + 1, (ci + 1) % 2)
            stage_out(ci, ci % 2, descs)
            descs = nxt

    return k(xf, weight, opat, bias)


def kernel(x, weight, bias, offset):
    opat = jnp.tile(offset.astype(jnp.int32), PER // F)
    return _field_linear_sc(x.reshape(-1), weight, opat,
                            bias.astype(jnp.float32))


# speedup vs baseline: 1.0196x; 1.0196x over previous
import functools

import jax
import jax.numpy as jnp
from jax import lax
from jax.experimental import pallas as pl
from jax.experimental.pallas import tpu as pltpu
from jax.experimental.pallas import tpu_sc as plsc

F = 26
OUT = 16
B = 16384
NW = 32
BPT = B // NW
C = 128
NCHUNK = BPT // C
NV = C // 16


def _field_linear_sc(xt, weight, off2, bias):
    mesh = plsc.VectorSubcoreMesh(core_axis_name="c", subcore_axis_name="s")

    @functools.partial(
        pl.kernel,
        out_type=jax.ShapeDtypeStruct((B, OUT), jnp.float32),
        mesh=mesh,
        compiler_params=pltpu.CompilerParams(use_tc_tiling_on_sc=False),
        scratch_types=[
            pltpu.VMEM((F, OUT), jnp.int32),
            pltpu.VMEM((OUT,), jnp.float32),
            pltpu.VMEM((2, F, C), jnp.int32),
            pltpu.VMEM((2, F, C), jnp.int32),
            pltpu.VMEM((2, F, C, OUT), jnp.float32),
            pltpu.VMEM((C, OUT), jnp.float32),
            pltpu.SemaphoreType.DMA,
            pltpu.SemaphoreType.DMA,
        ],
    )
    def k(xt_hbm, w_hbm, off2_hbm, bias_hbm, out_hbm,
          off2_v, bias_v, xv, idx_v, gbuf, outb, sem0, sem1):
        cid = lax.axis_index("c")
        sid = lax.axis_index("s")
        wid = sid * 2 + cid
        tbase = wid * BPT
        sems = (sem0, sem1)

        pltpu.sync_copy(off2_hbm, off2_v)
        pltpu.sync_copy(bias_hbm, bias_v)
        bias_vec = bias_v[:]

        def stage_in(ci, pb):
            base = tbase + ci * C
            pltpu.sync_copy(xt_hbm.at[:, pl.ds(base, C)], xv.at[pb])

            def vbody(j, carry):
                s = pl.ds(j * 16, 16)
                for f in range(F):
                    idx_v[pb, f, s] = xv[pb, f, s] + off2_v[f, :]
                return carry

            lax.fori_loop(0, NV, vbody, 0)
            return [
                pltpu.async_copy(w_hbm.at[idx_v.at[pb, f]], gbuf.at[pb, f],
                                 sems[pb])
                for f in range(F)
            ]

        def stage_out(ci, pb, descs):
            for dsc in descs:
                dsc.wait()

            def row_body(j, carry):
                acc = bias_vec
                for f in range(F):
                    acc = acc + gbuf[pb, f, j, :]
                outb[j, :] = acc
                return carry

            lax.fori_loop(0, C, row_body, 0)
            base = tbase + ci * C
            pltpu.sync_copy(outb, out_hbm.at[pl.ds(base, C), :])

        descs = stage_in(0, 0)
        for ci in range(NCHUNK):
            nxt = None
            if ci + 1 < NCHUNK:
                nxt = stage_in(ci + 1, (ci + 1) % 2)
            stage_out(ci, ci % 2, descs)
            descs = nxt

    return k(xt, weight, off2, bias)


def kernel(x, weight, bias, offset):
    off2 = jnp.broadcast_to(offset.astype(jnp.int32)[:, None], (F, OUT))
    return _field_linear_sc(x.T, weight, off2, bias.astype(jnp.float32))
